# baseline (speedup 1.0000x reference)
import functools

import jax
import jax.numpy as jnp
from jax.experimental import pallas as pl
from jax.experimental.pallas import tpu as pltpu

_C = 384
_HF = 64
_WF = 64
_N = _HF * _WF
_K = 20
_SBLK = 512
_GRID = _N // _SBLK
_NITER = 16
_S = 30000.0


def _body(q_ref, s_ref, m_ref, fg_ref, bg_ref, qn_ref):
    @pl.when(pl.program_id(0) == 0)
    def _():
        q = q_ref[...]
        qss = jnp.sum(q * q, axis=0, keepdims=True)
        qn_ref[...] = (q * (1.0 / jnp.maximum(jnp.sqrt(qss), 1e-12))
                       ).astype(jnp.bfloat16)

    qn = qn_ref[...]
    s = s_ref[...]
    sss = jnp.sum(s * s, axis=0, keepdims=True)
    sn = (s * (1.0 / jnp.maximum(jnp.sqrt(sss), 1e-12))).astype(jnp.bfloat16)
    sim = jax.lax.dot_general(
        qn, sn, (((0,), (0,)), ((), ())),
        preferred_element_type=jnp.float32)

    m = m_ref[...] > 0.5
    msum = jnp.sum(m_ref[...])
    kf = jnp.minimum(msum, float(_K))
    kb = jnp.minimum(float(_N) - msum, float(_K))
    kfi = kf.astype(jnp.int32)
    kbi = kb.astype(jnp.int32)

    m16 = m_ref[...].astype(jnp.int16)
    simi = (sim * _S).astype(jnp.int16)

    lo_f = jnp.full((1, _SBLK), -30400, jnp.int32)
    hi_f = jnp.full((1, _SBLK), 30400, jnp.int32)
    lo_b = lo_f
    hi_b = hi_f
    one16 = jnp.ones((_N, 1), jnp.int16)
    mb16 = one16 - m16

    def _treesum(x):
        n = x.shape[0]
        while n > 16:
            h = n // 2
            x = x[:h] + x[h:]
            n = h
        return jnp.sum(x.astype(jnp.int32), axis=0, keepdims=True)

    chf = jnp.zeros((1, _SBLK), jnp.int32)
    chb = chf
    for _ in range(_NITER):
        mid_f = (lo_f + hi_f) >> 1
        mid_b = (lo_b + hi_b) >> 1
        gtf = simi > mid_f.astype(jnp.int16)
        cnt_f = _treesum(jnp.where(gtf, m16, 0))
        gtb = simi > mid_b.astype(jnp.int16)
        cnt_b = _treesum(jnp.where(gtb, mb16, 0))
        pf = cnt_f > kfi
        lo_f = jnp.where(pf, mid_f, lo_f)
        hi_f = jnp.where(pf, hi_f, mid_f)
        chf = jnp.where(pf, chf, cnt_f)
        pb = cnt_b > kbi
        lo_b = jnp.where(pb, mid_b, lo_b)
        hi_b = jnp.where(pb, hi_b, mid_b)
        chb = jnp.where(pb, chb, cnt_b)

    def _treesum32(x):
        n = x.shape[0]
        while n > 8:
            h = n // 2
            x = x[:h] + x[h:]
            n = h
        return jnp.sum(x, axis=0, keepdims=True)

    tf = (hi_f.astype(jnp.float32) + 0.5) * (1.0 / _S)
    tb = (hi_b.astype(jnp.float32) + 0.5) * (1.0 / _S)
    simf = jnp.where(m, sim, 0.0)
    gtf = simi > hi_f.astype(jnp.int16)
    gtb = simi > hi_b.astype(jnp.int16)
    sum_f = _treesum32(jnp.where(gtf, simf, 0.0))
    sum_b = _treesum32(jnp.where(gtb, sim - simf, 0.0))
    cnt_f = chf.astype(jnp.float32)
    cnt_b = chb.astype(jnp.float32)
    tot_f = sum_f + (kf - cnt_f) * tf
    tot_b = sum_b + (kb - cnt_b) * tb
    fg = tot_f / jnp.maximum(kf, 1.0)
    bg = tot_b / jnp.maximum(kb, 1.0)
    fg_ref[...] = jnp.where(kf > 0.0, fg, 0.0)
    bg_ref[...] = jnp.where(kb > 0.0, bg, 0.0)


@functools.partial(jax.jit, static_argnames=("interpret",))
def _run(q2, s2, maskf, interpret=False):
    fg, bg = pl.pallas_call(
        _body,
        grid=(_GRID,),
        in_specs=[
            pl.BlockSpec((_C, _N), lambda i: (0, 0)),
            pl.BlockSpec((_C, _SBLK), lambda i: (0, i)),
            pl.BlockSpec((_N, 1), lambda i: (0, 0)),
        ],
        out_specs=[
            pl.BlockSpec((1, _SBLK), lambda i: (0, i)),
            pl.BlockSpec((1, _SBLK), lambda i: (0, i)),
        ],
        out_shape=[
            jax.ShapeDtypeStruct((1, _N), jnp.float32),
            jax.ShapeDtypeStruct((1, _N), jnp.float32),
        ],
        scratch_shapes=[pltpu.VMEM((_C, _N), jnp.bfloat16)],
        interpret=interpret,
    )(q2, s2, maskf)
    return fg, bg


def kernel(query_label, color, q_feat, s_feat, object_index):
    q2 = q_feat.reshape(_C, _N)
    s2 = s_feat.reshape(_C, _N)
    maskf = jnp.all(query_label == color, axis=-1).reshape(_N, 1)
    maskf = maskf.astype(jnp.float32)
    fg, bg = _run(q2, s2, maskf)
    return (fg.reshape(_HF, _WF), bg.reshape(_HF, _WF))


# speedup vs baseline: 1.1383x; 1.1383x over previous
import functools

import jax
import jax.numpy as jnp
from jax.experimental import pallas as pl
from jax.experimental.pallas import tpu as pltpu

_C = 384
_HF = 64
_WF = 64
_N = _HF * _WF
_K = 20
_SBLK = 512
_GRID = _N // _SBLK
_NITER = 16
_S = 30000.0


def _body(q_ref, s_ref, m_ref, fg_ref, bg_ref, qn_ref):
    @pl.when(pl.program_id(0) == 0)
    def _():
        q = q_ref[...]
        qss = jnp.sum(q * q, axis=0, keepdims=True)
        qn_ref[...] = (q * (1.0 / jnp.maximum(jnp.sqrt(qss), 1e-12))
                       ).astype(jnp.bfloat16)

    qn = qn_ref[...]
    s = s_ref[...]
    sss = jnp.sum(s * s, axis=0, keepdims=True)
    sn = (s * (1.0 / jnp.maximum(jnp.sqrt(sss), 1e-12))).astype(jnp.bfloat16)
    sim = jax.lax.dot_general(
        qn, sn, (((0,), (0,)), ((), ())),
        preferred_element_type=jnp.float32)

    m = m_ref[...] > 0.5
    msum = jnp.sum(m_ref[...])
    kf = jnp.minimum(msum, float(_K))
    kb = jnp.minimum(float(_N) - msum, float(_K))
    kfi = kf.astype(jnp.int32)
    kbi = kb.astype(jnp.int32)

    m16 = m_ref[...].astype(jnp.int16)
    simi = (sim * _S).astype(jnp.int16)

    lo_f = jnp.full((1, _SBLK), -30400, jnp.int32)
    hi_f = jnp.full((1, _SBLK), 30400, jnp.int32)
    lo_b = lo_f
    hi_b = hi_f
    one16 = jnp.ones((_N, 1), jnp.int16)
    mb16 = one16 - m16

    def _treesum(x):
        n = x.shape[0]
        while n > 16:
            h = n // 2
            x = x[:h] + x[h:]
            n = h
        return jnp.sum(x.astype(jnp.int32), axis=0, keepdims=True)

    chf = jnp.zeros((1, _SBLK), jnp.int32)
    chb = chf
    for _ in range(_NITER):
        mid_f = (lo_f + hi_f) >> 1
        mid_b = (lo_b + hi_b) >> 1
        gtf = simi > mid_f.astype(jnp.int16)
        cnt_f = _treesum(jnp.where(gtf, m16, 0))
        gtb = simi > mid_b.astype(jnp.int16)
        cnt_b = _treesum(jnp.where(gtb, mb16, 0))
        pf = cnt_f > kfi
        lo_f = jnp.where(pf, mid_f, lo_f)
        hi_f = jnp.where(pf, hi_f, mid_f)
        chf = jnp.where(pf, chf, cnt_f)
        pb = cnt_b > kbi
        lo_b = jnp.where(pb, mid_b, lo_b)
        hi_b = jnp.where(pb, hi_b, mid_b)
        chb = jnp.where(pb, chb, cnt_b)

    tf = (hi_f.astype(jnp.float32) + 0.5) * (1.0 / _S)
    tb = (hi_b.astype(jnp.float32) + 0.5) * (1.0 / _S)
    thr = jnp.where(m, tf, tb)
    gt = sim > thr
    vsel = jnp.where(gt, sim, 0.0)
    vf = jnp.where(m, vsel, 0.0)
    sum_f = jnp.sum(vf, axis=0, keepdims=True)
    sum_b = jnp.sum(vsel - vf, axis=0, keepdims=True)
    cnt_f = chf.astype(jnp.float32)
    cnt_b = chb.astype(jnp.float32)
    tot_f = sum_f + (kf - cnt_f) * tf
    tot_b = sum_b + (kb - cnt_b) * tb
    fg = tot_f / jnp.maximum(kf, 1.0)
    bg = tot_b / jnp.maximum(kb, 1.0)
    fg_ref[...] = jnp.where(kf > 0.0, fg, 0.0)
    bg_ref[...] = jnp.where(kb > 0.0, bg, 0.0)


@functools.partial(jax.jit, static_argnames=("interpret",))
def _run(q2, s2, maskf, interpret=False):
    fg, bg = pl.pallas_call(
        _body,
        grid=(_GRID,),
        in_specs=[
            pl.BlockSpec((_C, _N), lambda i: (0, 0)),
            pl.BlockSpec((_C, _SBLK), lambda i: (0, i)),
            pl.BlockSpec((_N, 1), lambda i: (0, 0)),
        ],
        out_specs=[
            pl.BlockSpec((1, _SBLK), lambda i: (0, i)),
            pl.BlockSpec((1, _SBLK), lambda i: (0, i)),
        ],
        out_shape=[
            jax.ShapeDtypeStruct((1, _N), jnp.float32),
            jax.ShapeDtypeStruct((1, _N), jnp.float32),
        ],
        scratch_shapes=[pltpu.VMEM((_C, _N), jnp.bfloat16)],
        interpret=interpret,
    )(q2, s2, maskf)
    return fg, bg


def kernel(query_label, color, q_feat, s_feat, object_index):
    q2 = q_feat.reshape(_C, _N)
    s2 = s_feat.reshape(_C, _N)
    maskf = jnp.all(query_label == color, axis=-1).reshape(_N, 1)
    maskf = maskf.astype(jnp.float32)
    fg, bg = _run(q2, s2, maskf)
    return (fg.reshape(_HF, _WF), bg.reshape(_HF, _WF))
